# baseline (speedup 1.0000x reference)
import functools
import math

import jax
import jax.numpy as jnp
from jax import lax
from jax.experimental import pallas as pl
from jax.experimental.pallas import tpu as pltpu
from jax.experimental.pallas import tpu_sc as plsc

N = 10000
E = 320000
NS = 16
CH = 128
NCHK = 168
HCH = (96, 72)
EPAD = NS * NCHK * CH
FAKE_DST = N + 4
HALF = 5120
ACCR = HALF + 8
ZR = 40
LEAK = 0.02
BN_C = 1.0 / math.sqrt(1.0 + 1e-5)

_MESH = dict(core_axis_name="c", subcore_axis_name="s")


def _build_deg():
    @functools.partial(
        pl.kernel,
        mesh=plsc.VectorSubcoreMesh(**_MESH),
        out_type=jax.ShapeDtypeStruct((N,), jnp.float32),
        scratch_types=[
            pltpu.VMEM((NCHK, CH), jnp.int32),
            pltpu.VMEM((128,), jnp.float32),
            pltpu.VMEM((1024,), jnp.float32),
            pltpu.VMEM((1000,), jnp.float32),
            pltpu.VMEM_SHARED((N + 8,), jnp.float32),
        ],
    )
    def deg_kernel(dst_hbm, deg_hbm, idx_v, ones_v, zer_v, tmp_v, acc_sh):
        c = lax.axis_index("c")
        s = lax.axis_index("s")

        @pl.when(c == 0)
        def _():
            def fill_ones(i, _):
                ones_v[pl.ds(i * 16, 16)] = jnp.ones((16,), jnp.float32)
                return 0

            lax.fori_loop(0, 8, fill_ones, 0)

            def fill_zer(i, _):
                zer_v[pl.ds(i * 16, 16)] = jnp.zeros((16,), jnp.float32)
                return 0

            lax.fori_loop(0, 64, fill_zer, 0)

            @pl.when(s < 10)
            def _():
                pltpu.sync_copy(zer_v.at[pl.ds(0, 1000)],
                                acc_sh.at[pl.ds(s * 1000, 1000)])

            plsc.subcore_barrier()
            pltpu.sync_copy(dst_hbm.at[s], idx_v)

            def step(j, _):
                pltpu.sync_copy(ones_v.at[pl.ds(0, CH)],
                                acc_sh.at[idx_v.at[j]], add=True)
                return 0

            lax.fori_loop(0, NCHK, step, 0)
            plsc.subcore_barrier()

            @pl.when(s < 10)
            def _():
                pltpu.sync_copy(acc_sh.at[pl.ds(s * 1000, 1000)], tmp_v)
                pltpu.sync_copy(tmp_v, deg_hbm.at[pl.ds(s * 1000, 1000)])

    return deg_kernel


def _zero_acc(s, zer_v, acc_sh):
    def fill_zer(i, _):
        zer_v[i // 8, pl.ds((i % 8) * 16, 16)] = jnp.zeros((16,), jnp.float32)
        return 0

    lax.fori_loop(0, ZR * 8, fill_zer, 0)

    def zero_blk(t, _):
        pltpu.sync_copy(zer_v.at[pl.ds(0, ZR)],
                        acc_sh.at[pl.ds(s * 8 * ZR + t * ZR, ZR)])
        return 0

    lax.fori_loop(0, 8, zero_blk, 0)


def _remap_dst(c, dst_v, ln):
    base = c * HALF

    def remap(i, _):
        v = dst_v[i // 8, pl.ds((i % 8) * 16, 16)] - base
        ok = (v >= 0) & (v < HALF + 8)
        dst_v[i // 8, pl.ds((i % 8) * 16, 16)] = jnp.where(
            ok, v, HALF + (v & 7))
        return 0

    lax.fori_loop(0, ln * 8, remap, 0)


def _accumulate1(z_hbm, src_v, dst_v, rows_v, acc_sh, sem):
    def step(j, _):
        pltpu.async_copy(z_hbm.at[src_v.at[j]], rows_v, sem).wait()
        pltpu.sync_copy(rows_v, acc_sh.at[dst_v.at[j]], add=True)
        return 0

    lax.fori_loop(0, NCHK, step, 0)


def _run_half(c, s, z_hbm, src_hbm, dst_hbm, src_v, dst_v, bufs, acc_sh,
              sems, st, ln):
    pltpu.sync_copy(src_hbm.at[s].at[pl.ds(st, ln)], src_v.at[pl.ds(0, ln)])
    pltpu.sync_copy(dst_hbm.at[s].at[pl.ds(st, ln)], dst_v.at[pl.ds(0, ln)])
    _remap_dst(c, dst_v, ln)
    for t in range(2):
        pltpu.async_copy(z_hbm.at[src_v.at[t]], bufs[t], sems[t])

    def triple(i, _):
        j = i * 3
        for t in range(3):
            jj = j + t

            @pl.when(jj + 2 < ln)
            def _(t=t, jj=jj):
                pltpu.async_copy(z_hbm.at[src_v.at[jj + 2]],
                                 bufs[(t + 2) % 3], sems[(t + 2) % 3])

            pltpu.make_async_copy(z_hbm.at[src_v.at[jj]], bufs[t],
                                  sems[t]).wait()
            pltpu.sync_copy(bufs[t], acc_sh.at[dst_v.at[jj]], add=True)
        return 0

    lax.fori_loop(0, ln // 3, triple, 0)


def _accumulate(c, s, z_hbm, src_hbm, dst_hbm, src_v, dst_v, bufs, acc_sh,
                sems):
    for st, ln in ((0, HCH[0]), (HCH[0], HCH[1])):
        _run_half(c, s, z_hbm, src_hbm, dst_hbm, src_v, dst_v, bufs,
                  acc_sh, sems, st, ln)


def _writeback(c, s, acc_sh, out_hbm):
    @pl.when(c == 0)
    def _():
        pltpu.sync_copy(acc_sh.at[pl.ds(s * 320, 320)],
                        out_hbm.at[pl.ds(s * 320, 320)])

    @pl.when((c == 1) & (s < 12))
    def _():
        pltpu.sync_copy(acc_sh.at[pl.ds(s * 400, 400)],
                        out_hbm.at[pl.ds(HALF + s * 400, 400)])

    @pl.when((c == 1) & (s == 12))
    def _():
        pltpu.sync_copy(acc_sh.at[pl.ds(4800, 80)],
                        out_hbm.at[pl.ds(HALF + 4800, 80)])


def _prop_scratch():
    return [
        pltpu.VMEM((HCH[0], CH), jnp.int32),
        pltpu.VMEM((HCH[0], CH), jnp.int32),
        pltpu.VMEM_SHARED((ACCR, 128), jnp.float32),
    ] + [pltpu.VMEM((CH, 128), jnp.float32) for _ in range(3)] \
      + [pltpu.SemaphoreType.DMA for _ in range(3)]


def _build_prop1():
    @functools.partial(
        pl.kernel,
        mesh=plsc.VectorSubcoreMesh(**_MESH),
        out_type=jax.ShapeDtypeStruct((N, 128), jnp.float32),
        scratch_types=_prop_scratch(),
    )
    def prop_kernel(src_hbm, dst_hbm, z_hbm, out_hbm,
                    src_v, dst_v, acc_sh, b0, b1, b2, s0, s1, s2):
        c = lax.axis_index("c")
        s = lax.axis_index("s")
        _zero_acc(s, b0, acc_sh)
        plsc.subcore_barrier()
        _accumulate(c, s, z_hbm, src_hbm, dst_hbm, src_v, dst_v,
                    [b0, b1, b2], acc_sh, [s0, s1, s2])
        plsc.subcore_barrier()
        _writeback(c, s, acc_sh, out_hbm)

    return prop_kernel


def _build_prop2():
    @functools.partial(
        pl.kernel,
        mesh=plsc.VectorSubcoreMesh(**_MESH),
        out_type=[
            jax.ShapeDtypeStruct((N, 128), jnp.float32),
            jax.ShapeDtypeStruct((N, 128), jnp.float32),
        ],
        scratch_types=_prop_scratch(),
    )
    def prop_kernel(src_hbm, dst_hbm, za_hbm, zb_hbm, outa_hbm, outb_hbm,
                    src_v, dst_v, acc_sh, b0, b1, b2, s0, s1, s2):
        c = lax.axis_index("c")
        s = lax.axis_index("s")
        _zero_acc(s, b0, acc_sh)
        plsc.subcore_barrier()
        _accumulate(c, s, za_hbm, src_hbm, dst_hbm, src_v, dst_v,
                    [b0, b1, b2], acc_sh, [s0, s1, s2])
        plsc.subcore_barrier()
        _writeback(c, s, acc_sh, outa_hbm)
        plsc.subcore_barrier()
        _zero_acc(s, b0, acc_sh)
        plsc.subcore_barrier()
        _accumulate(c, s, zb_hbm, src_hbm, dst_hbm, src_v, dst_v,
                    [b0, b1, b2], acc_sh, [s0, s1, s2])
        plsc.subcore_barrier()
        _writeback(c, s, acc_sh, outb_hbm)

    return prop_kernel


_deg = _build_deg()
_prop1 = _build_prop1()
_prop2 = _build_prop2()


R = 1000
G = N // R


def _row_spec(w):
    return pl.BlockSpec((R, w), lambda i: (i, 0))


def _full_spec(a, b):
    return pl.BlockSpec((a, b), lambda i: (0, 0))


def _tc1(deg2, x, W1):
    def body(deg_ref, x_ref, w_ref, dis_ref, z1_ref):
        dis = lax.rsqrt(deg_ref[...] + 1.0)
        z = jnp.dot(x_ref[...], w_ref[...],
                    preferred_element_type=jnp.float32) * dis
        dis_ref[...] = dis
        z1_ref[...] = jnp.concatenate([z, jnp.zeros_like(z)], axis=1)

    return pl.pallas_call(
        body,
        grid=(G,),
        in_specs=[_row_spec(1), _row_spec(128), _full_spec(128, 64)],
        out_specs=[_row_spec(1), _row_spec(128)],
        out_shape=[
            jax.ShapeDtypeStruct((N, 1), jnp.float32),
            jax.ShapeDtypeStruct((N, 128), jnp.float32),
        ],
    )(deg2, x, W1)


def _tc2(a1, z1, dis2, b1r, g1r, be1r, W2):
    def body(a1_ref, z1_ref, dis_ref, b_ref, g_ref, be_ref, w_ref, z2_ref):
        a = (a1_ref[...] + z1_ref[...])[:, :64]
        dis = dis_ref[...]
        t = (dis * a + b_ref[...]) * (g_ref[...] * BN_C) + be_ref[...]
        h1 = jnp.where(t >= 0.0, t, LEAK * t)
        z2_ref[...] = jnp.dot(h1, w_ref[...],
                              preferred_element_type=jnp.float32) * dis

    return pl.pallas_call(
        body,
        grid=(G,),
        in_specs=[_row_spec(128), _row_spec(128), _row_spec(1),
                  _full_spec(1, 64), _full_spec(1, 64), _full_spec(1, 64),
                  _full_spec(64, 128)],
        out_specs=_row_spec(128),
        out_shape=jax.ShapeDtypeStruct((N, 128), jnp.float32),
    )(a1, z1, dis2, b1r, g1r, be1r, W2)


def _tc3(a2, z2, dis2, b2r, g2r, be2r, W3):
    def body(a2_ref, z2_ref, dis_ref, b_ref, g_ref, be_ref, w_ref,
             xrep_ref, z3lo_ref, z3hi_ref):
        a = a2_ref[...] + z2_ref[...]
        dis = dis_ref[...]
        xr = (dis * a + b_ref[...]) * (g_ref[...] * BN_C) + be_ref[...]
        xrep_ref[...] = xr
        z3 = jnp.dot(xr, w_ref[...],
                     preferred_element_type=jnp.float32) * dis
        z3lo_ref[...] = z3[:, :128]
        z3hi_ref[...] = z3[:, 128:]

    return pl.pallas_call(
        body,
        grid=(G,),
        in_specs=[_row_spec(128), _row_spec(128), _row_spec(1),
                  _full_spec(1, 128), _full_spec(1, 128), _full_spec(1, 128),
                  _full_spec(128, 256)],
        out_specs=[_row_spec(128), _row_spec(128), _row_spec(128)],
        out_shape=[
            jax.ShapeDtypeStruct((N, 128), jnp.float32),
            jax.ShapeDtypeStruct((N, 128), jnp.float32),
            jax.ShapeDtypeStruct((N, 128), jnp.float32),
        ],
    )(a2, z2, dis2, b2r, g2r, be2r, W3)


def _tc4(a3lo, a3hi, zlo, zhi, dis2, b3r):
    def body(alo_ref, ahi_ref, zlo_ref, zhi_ref, dis_ref, b_ref, xemb_ref):
        a = jnp.concatenate(
            [alo_ref[...] + zlo_ref[...], ahi_ref[...] + zhi_ref[...]],
            axis=1)
        xemb_ref[...] = dis_ref[...] * a + b_ref[...]

    return pl.pallas_call(
        body,
        grid=(G,),
        in_specs=[_row_spec(128), _row_spec(128), _row_spec(128),
                  _row_spec(128), _row_spec(1), _full_spec(1, 256)],
        out_specs=_row_spec(256),
        out_shape=jax.ShapeDtypeStruct((N, 256), jnp.float32),
    )(a3lo, a3hi, zlo, zhi, dis2, b3r)


def kernel(x, edge_index, W1, b1, g1, be1, W2, b2, g2, be2, W3, b3, Wfc, bfc):
    pad = EPAD - E
    src = jnp.concatenate(
        [edge_index[0], jnp.zeros((pad,), jnp.int32)]).reshape(NS, NCHK, CH)
    dst = jnp.concatenate(
        [edge_index[1],
         jnp.full((pad,), FAKE_DST, jnp.int32)]).reshape(NS, NCHK, CH)

    deg = _deg(dst)
    deg2 = deg.reshape(N, 1)

    dis2, z1 = _tc1(deg2, x, W1)
    a1 = _prop1(src, dst, z1)
    z2 = _tc2(a1, z1, dis2,
              b1.reshape(1, 64), g1.reshape(1, 64), be1.reshape(1, 64), W2)
    a2 = _prop1(src, dst, z2)
    x_rep, z3lo, z3hi = _tc3(a2, z2, dis2,
                             b2.reshape(1, 128), g2.reshape(1, 128),
                             be2.reshape(1, 128), W3)
    a3lo, a3hi = _prop2(src, dst, z3lo, z3hi)
    x_emb = _tc4(a3lo, a3hi, z3lo, z3hi, dis2, b3.reshape(1, 256))
    return (x_rep, x_emb)


# speedup vs baseline: 4.1648x; 4.1648x over previous
import functools
import math

import jax
import jax.numpy as jnp
from jax import lax
from jax.experimental import pallas as pl
from jax.experimental.pallas import tpu as pltpu
from jax.experimental.pallas import tpu_sc as plsc

N = 10000
E = 320000
NS = 16
CH = 128
NCHK = 158
EPAD = NS * NCHK * CH
FAKE_DST = N + 8
HALF = 5120
ACCR = HALF + 64
ZR = 40
LEAK = 0.02
BN_C = 1.0 / math.sqrt(1.0 + 1e-5)

_MESH = dict(core_axis_name="c", subcore_axis_name="s")


def _build_deg():
    @functools.partial(
        pl.kernel,
        mesh=plsc.VectorSubcoreMesh(**_MESH),
        out_type=jax.ShapeDtypeStruct((N,), jnp.float32),
        scratch_types=[
            pltpu.VMEM((NCHK, CH), jnp.int32),
            pltpu.VMEM((128,), jnp.float32),
            pltpu.VMEM((1024,), jnp.float32),
            pltpu.VMEM((1000,), jnp.float32),
            pltpu.VMEM_SHARED((N + 16,), jnp.float32),
        ],
    )
    def deg_kernel(dst_hbm, deg_hbm, idx_v, ones_v, zer_v, tmp_v, acc_sh):
        c = lax.axis_index("c")
        s = lax.axis_index("s")

        @pl.when(c == 0)
        def _():
            def fill_ones(i, _):
                ones_v[pl.ds(i * 16, 16)] = jnp.ones((16,), jnp.float32)
                return 0

            lax.fori_loop(0, 8, fill_ones, 0)

            def fill_zer(i, _):
                zer_v[pl.ds(i * 16, 16)] = jnp.zeros((16,), jnp.float32)
                return 0

            lax.fori_loop(0, 64, fill_zer, 0)

            @pl.when(s < 10)
            def _():
                pltpu.sync_copy(zer_v.at[pl.ds(0, 1000)],
                                acc_sh.at[pl.ds(s * 1000, 1000)])

            plsc.subcore_barrier()
            pltpu.sync_copy(dst_hbm.at[s], idx_v)

            def step(j, _):
                pltpu.sync_copy(ones_v.at[pl.ds(0, CH)],
                                acc_sh.at[idx_v.at[j]], add=True)
                return 0

            lax.fori_loop(0, NCHK, step, 0)
            plsc.subcore_barrier()

            @pl.when(s < 10)
            def _():
                pltpu.sync_copy(acc_sh.at[pl.ds(s * 1000, 1000)], tmp_v)
                pltpu.sync_copy(tmp_v, deg_hbm.at[pl.ds(s * 1000, 1000)])

    return deg_kernel


def _zero_acc(s, zer_v, acc_sh):
    def fill_zer(i, _):
        zer_v[i // 8, pl.ds((i % 8) * 16, 16)] = jnp.zeros((16,), jnp.float32)
        return 0

    lax.fori_loop(0, ZR * 8, fill_zer, 0)

    def zero_blk(t, _):
        pltpu.sync_copy(zer_v, acc_sh.at[pl.ds(s * 8 * ZR + t * ZR, ZR)])
        return 0

    lax.fori_loop(0, 8, zero_blk, 0)


def _remap_dst(c, dst_v):
    base = c * HALF

    def remap(i, _):
        v = dst_v[i // 8, pl.ds((i % 8) * 16, 16)] - base
        ok = (v >= 0) & (v < HALF + 8)
        dst_v[i // 8, pl.ds((i % 8) * 16, 16)] = jnp.where(
            ok, v, HALF + (v & 63))
        return 0

    lax.fori_loop(0, NCHK * 8, remap, 0)


def _accumulate1(z_hbm, src_v, dst_v, rows_v, acc_sh, sem):
    def step(j, _):
        pltpu.async_copy(z_hbm.at[src_v.at[j]], rows_v, sem).wait()
        pltpu.sync_copy(rows_v, acc_sh.at[dst_v.at[j]], add=True)
        return 0

    lax.fori_loop(0, NCHK, step, 0)


def _accumulate(z_hbm, src_v, dst_v, rows_a, rows_b, acc_sh, sem_a, sem_b):
    pltpu.async_copy(z_hbm.at[src_v.at[0]], rows_a, sem_a)

    def pair(i, _):
        j = i * 2
        pltpu.async_copy(z_hbm.at[src_v.at[j + 1]], rows_b, sem_b)
        pltpu.make_async_copy(z_hbm.at[src_v.at[j]], rows_a, sem_a).wait()
        pltpu.sync_copy(rows_a, acc_sh.at[dst_v.at[j]], add=True)

        @pl.when(j + 2 < NCHK)
        def _():
            pltpu.async_copy(z_hbm.at[src_v.at[j + 2]], rows_a, sem_a)

        pltpu.make_async_copy(z_hbm.at[src_v.at[j + 1]], rows_b, sem_b).wait()
        pltpu.sync_copy(rows_b, acc_sh.at[dst_v.at[j + 1]], add=True)
        return 0

    lax.fori_loop(0, NCHK // 2, pair, 0)


def _writeback(c, s, acc_sh, out_hbm):
    @pl.when(c == 0)
    def _():
        pltpu.sync_copy(acc_sh.at[pl.ds(s * 320, 320)],
                        out_hbm.at[pl.ds(s * 320, 320)])

    @pl.when((c == 1) & (s < 12))
    def _():
        pltpu.sync_copy(acc_sh.at[pl.ds(s * 400, 400)],
                        out_hbm.at[pl.ds(HALF + s * 400, 400)])

    @pl.when((c == 1) & (s == 12))
    def _():
        pltpu.sync_copy(acc_sh.at[pl.ds(4800, 80)],
                        out_hbm.at[pl.ds(HALF + 4800, 80)])


def _prop_scratch(nbuf):
    return [
        pltpu.VMEM((NCHK, CH), jnp.int32),
        pltpu.VMEM((NCHK, CH), jnp.int32),
        pltpu.VMEM((ZR, 128), jnp.float32),
        pltpu.VMEM_SHARED((ACCR, 128), jnp.float32),
    ] + [pltpu.VMEM((CH, 128), jnp.float32)] * nbuf \
      + [pltpu.SemaphoreType.DMA] * nbuf


def _build_prop1():
    @functools.partial(
        pl.kernel,
        mesh=plsc.VectorSubcoreMesh(**_MESH),
        out_type=jax.ShapeDtypeStruct((N, 128), jnp.float32),
        scratch_types=_prop_scratch(2),
    )
    def prop_kernel(src_hbm, dst_hbm, z_hbm, out_hbm,
                    src_v, dst_v, zer_v, acc_sh, rows_a, rows_b,
                    sem_a, sem_b):
        c = lax.axis_index("c")
        s = lax.axis_index("s")
        _zero_acc(s, zer_v, acc_sh)
        pltpu.sync_copy(src_hbm.at[s], src_v)
        pltpu.sync_copy(dst_hbm.at[s], dst_v)
        _remap_dst(c, dst_v)
        plsc.subcore_barrier()
        _accumulate(z_hbm, src_v, dst_v, rows_a, rows_b, acc_sh,
                    sem_a, sem_b)
        plsc.subcore_barrier()
        _writeback(c, s, acc_sh, out_hbm)

    return prop_kernel


def _build_prop2():
    @functools.partial(
        pl.kernel,
        mesh=plsc.VectorSubcoreMesh(**_MESH),
        out_type=[
            jax.ShapeDtypeStruct((N, 128), jnp.float32),
            jax.ShapeDtypeStruct((N, 128), jnp.float32),
        ],
        scratch_types=_prop_scratch(2),
    )
    def prop_kernel(src_hbm, dst_hbm, za_hbm, zb_hbm, outa_hbm, outb_hbm,
                    src_v, dst_v, zer_v, acc_sh, rows_a, rows_b,
                    sem_a, sem_b):
        c = lax.axis_index("c")
        s = lax.axis_index("s")
        _zero_acc(s, zer_v, acc_sh)
        pltpu.sync_copy(src_hbm.at[s], src_v)
        pltpu.sync_copy(dst_hbm.at[s], dst_v)
        _remap_dst(c, dst_v)
        plsc.subcore_barrier()
        _accumulate(za_hbm, src_v, dst_v, rows_a, rows_b, acc_sh,
                    sem_a, sem_b)
        plsc.subcore_barrier()
        _writeback(c, s, acc_sh, outa_hbm)
        plsc.subcore_barrier()
        _zero_acc(s, zer_v, acc_sh)
        plsc.subcore_barrier()
        _accumulate(zb_hbm, src_v, dst_v, rows_a, rows_b, acc_sh,
                    sem_a, sem_b)
        plsc.subcore_barrier()
        _writeback(c, s, acc_sh, outb_hbm)

    return prop_kernel


_deg = _build_deg()
_prop1 = _build_prop1()
_prop2 = _build_prop2()


R = 1000
G = N // R


def _row_spec(w):
    return pl.BlockSpec((R, w), lambda i: (i, 0))


def _full_spec(a, b):
    return pl.BlockSpec((a, b), lambda i: (0, 0))


def _tc1(deg2, x, W1):
    def body(deg_ref, x_ref, w_ref, dis_ref, z1_ref):
        dis = lax.rsqrt(deg_ref[...] + 1.0)
        z = jnp.dot(x_ref[...], w_ref[...],
                    preferred_element_type=jnp.float32) * dis
        dis_ref[...] = dis
        z1_ref[...] = jnp.concatenate([z, jnp.zeros_like(z)], axis=1)

    return pl.pallas_call(
        body,
        grid=(G,),
        in_specs=[_row_spec(1), _row_spec(128), _full_spec(128, 64)],
        out_specs=[_row_spec(1), _row_spec(128)],
        out_shape=[
            jax.ShapeDtypeStruct((N, 1), jnp.float32),
            jax.ShapeDtypeStruct((N, 128), jnp.float32),
        ],
    )(deg2, x, W1)


def _tc2(a1, z1, dis2, b1r, g1r, be1r, W2):
    def body(a1_ref, z1_ref, dis_ref, b_ref, g_ref, be_ref, w_ref, z2_ref):
        a = (a1_ref[...] + z1_ref[...])[:, :64]
        dis = dis_ref[...]
        t = (dis * a + b_ref[...]) * (g_ref[...] * BN_C) + be_ref[...]
        h1 = jnp.where(t >= 0.0, t, LEAK * t)
        z2_ref[...] = jnp.dot(h1, w_ref[...],
                              preferred_element_type=jnp.float32) * dis

    return pl.pallas_call(
        body,
        grid=(G,),
        in_specs=[_row_spec(128), _row_spec(128), _row_spec(1),
                  _full_spec(1, 64), _full_spec(1, 64), _full_spec(1, 64),
                  _full_spec(64, 128)],
        out_specs=_row_spec(128),
        out_shape=jax.ShapeDtypeStruct((N, 128), jnp.float32),
    )(a1, z1, dis2, b1r, g1r, be1r, W2)


def _tc3(a2, z2, dis2, b2r, g2r, be2r, W3):
    def body(a2_ref, z2_ref, dis_ref, b_ref, g_ref, be_ref, w_ref,
             xrep_ref, z3lo_ref, z3hi_ref):
        a = a2_ref[...] + z2_ref[...]
        dis = dis_ref[...]
        xr = (dis * a + b_ref[...]) * (g_ref[...] * BN_C) + be_ref[...]
        xrep_ref[...] = xr
        z3 = jnp.dot(xr, w_ref[...],
                     preferred_element_type=jnp.float32) * dis
        z3lo_ref[...] = z3[:, :128]
        z3hi_ref[...] = z3[:, 128:]

    return pl.pallas_call(
        body,
        grid=(G,),
        in_specs=[_row_spec(128), _row_spec(128), _row_spec(1),
                  _full_spec(1, 128), _full_spec(1, 128), _full_spec(1, 128),
                  _full_spec(128, 256)],
        out_specs=[_row_spec(128), _row_spec(128), _row_spec(128)],
        out_shape=[
            jax.ShapeDtypeStruct((N, 128), jnp.float32),
            jax.ShapeDtypeStruct((N, 128), jnp.float32),
            jax.ShapeDtypeStruct((N, 128), jnp.float32),
        ],
    )(a2, z2, dis2, b2r, g2r, be2r, W3)


def _tc4(a3lo, a3hi, zlo, zhi, dis2, b3r):
    def body(alo_ref, ahi_ref, zlo_ref, zhi_ref, dis_ref, b_ref, xemb_ref):
        a = jnp.concatenate(
            [alo_ref[...] + zlo_ref[...], ahi_ref[...] + zhi_ref[...]],
            axis=1)
        xemb_ref[...] = dis_ref[...] * a + b_ref[...]

    return pl.pallas_call(
        body,
        grid=(G,),
        in_specs=[_row_spec(128), _row_spec(128), _row_spec(128),
                  _row_spec(128), _row_spec(1), _full_spec(1, 256)],
        out_specs=_row_spec(256),
        out_shape=jax.ShapeDtypeStruct((N, 256), jnp.float32),
    )(a3lo, a3hi, zlo, zhi, dis2, b3r)


def kernel(x, edge_index, W1, b1, g1, be1, W2, b2, g2, be2, W3, b3, Wfc, bfc):
    pad = EPAD - E
    src = jnp.concatenate(
        [edge_index[0], jnp.zeros((pad,), jnp.int32)]).reshape(NS, NCHK, CH)
    dst = jnp.concatenate(
        [edge_index[1],
         jnp.full((pad,), FAKE_DST, jnp.int32)]).reshape(NS, NCHK, CH)

    deg = _deg(dst)
    deg2 = deg.reshape(N, 1)

    dis2, z1 = _tc1(deg2, x, W1)
    a1 = _prop1(src, dst, z1)
    z2 = _tc2(a1, z1, dis2,
              b1.reshape(1, 64), g1.reshape(1, 64), be1.reshape(1, 64), W2)
    a2 = _prop1(src, dst, z2)
    x_rep, z3lo, z3hi = _tc3(a2, z2, dis2,
                             b2.reshape(1, 128), g2.reshape(1, 128),
                             be2.reshape(1, 128), W3)
    a3lo, a3hi = _prop2(src, dst, z3lo, z3hi)
    x_emb = _tc4(a3lo, a3hi, z3lo, z3hi, dis2, b3.reshape(1, 256))
    return (x_rep, x_emb)
